# Initial kernel scaffold; baseline (speedup 1.0000x reference)
import jax
import jax.numpy as jnp
from jax.experimental import pallas as pl


def kernel(xyz, centers):
    raise NotImplementedError("write your pallas kernel here")


# speedup vs baseline: 73.6976x; 73.6976x over previous
import functools

import jax
import jax.numpy as jnp
from jax import lax
from jax.experimental import pallas as pl
from jax.experimental.pallas import tpu as pltpu
from jax.experimental.pallas import tpu_sc as plsc

KNN = 16
LANES = 16
NUM_WORKERS = 32


def _merge_sorted(run_v, run_i, new_v, new_i):
  sv, si = plsc.sort_key_val(new_v, new_i)
  sv = lax.rev(sv, (0,))
  si = lax.rev(si, (0,))
  keep = run_v <= sv
  lo_v = jnp.where(keep, run_v, sv)
  lo_i = jnp.where(keep, run_i, si)
  out_v, out_i = plsc.sort_key_val(lo_v, lo_i)
  return out_v, out_i


@functools.lru_cache(maxsize=None)
def _make_knn(b_sz, n_pts, k_cen):
  assert NUM_WORKERS % b_sz == 0
  workers_per_batch = NUM_WORKERS // b_sz
  cols_per_worker = k_cen // workers_per_batch
  assert cols_per_worker * workers_per_batch == k_cen
  blk_pts = 16 * LANES
  num_blocks = n_pts // blk_pts
  assert num_blocks * blk_pts == n_pts

  mesh = plsc.VectorSubcoreMesh(core_axis_name="c", subcore_axis_name="s")

  @functools.partial(
      pl.kernel,
      out_type=jax.ShapeDtypeStruct((b_sz * KNN, k_cen), jnp.int32),
      mesh=mesh,
      scratch_types=[
          pltpu.VMEM((n_pts,), jnp.float32),
          pltpu.VMEM((n_pts,), jnp.float32),
          pltpu.VMEM((n_pts,), jnp.float32),
          pltpu.VMEM((cols_per_worker,), jnp.float32),
          pltpu.VMEM((cols_per_worker,), jnp.float32),
          pltpu.VMEM((cols_per_worker,), jnp.float32),
          pltpu.VMEM((KNN, cols_per_worker), jnp.int32),
      ],
      compiler_params=pltpu.CompilerParams(needs_layout_passes=False),
  )
  def knn(x_h, y_h, z_h, cx_h, cy_h, cz_h, out_h,
          x_v, y_v, z_v, cx_v, cy_v, cz_v, out_v):
    wid = lax.axis_index("s") * 2 + lax.axis_index("c")
    b = wid // workers_per_batch
    c0 = (wid % workers_per_batch) * cols_per_worker

    pltpu.sync_copy(x_h.at[pl.ds(b * n_pts, n_pts)], x_v)
    pltpu.sync_copy(y_h.at[pl.ds(b * n_pts, n_pts)], y_v)
    pltpu.sync_copy(z_h.at[pl.ds(b * n_pts, n_pts)], z_v)
    pltpu.sync_copy(cx_h.at[pl.ds(b * k_cen + c0, cols_per_worker)], cx_v)
    pltpu.sync_copy(cy_h.at[pl.ds(b * k_cen + c0, cols_per_worker)], cy_v)
    pltpu.sync_copy(cz_h.at[pl.ds(b * k_cen + c0, cols_per_worker)], cz_v)

    lane_iota = lax.iota(jnp.int32, LANES)
    inf_v = jnp.full((LANES,), jnp.inf, jnp.float32)
    zero_i = jnp.zeros((LANES,), jnp.int32)

    @pl.loop(0, cols_per_worker)
    def col_loop(cl):
      cl_v = jnp.full((LANES,), cl, jnp.int32)
      cx = plsc.load_gather(cx_v, [cl_v])
      cy = plsc.load_gather(cy_v, [cl_v])
      cz = plsc.load_gather(cz_v, [cl_v])

      def blk_body(t, carry):
        run_v, run_i = carry
        base = t * blk_pts
        m = None
        for i in range(16):
          xv = x_v[pl.ds(base + i * LANES, LANES)]
          yv = y_v[pl.ds(base + i * LANES, LANES)]
          zv = z_v[pl.ds(base + i * LANES, LANES)]
          dx = xv - cx
          dy = yv - cy
          dz = zv - cz
          d2 = dx * dx + dy * dy + dz * dz
          m = d2 if m is None else jnp.minimum(m, d2)
        return _merge_sorted(run_v, run_i, m, lane_iota + t * LANES)

      _, cand = lax.fori_loop(0, num_blocks, blk_body, (inf_v, zero_i))

      pbase = ((cand >> 4) * blk_pts) + (cand & (LANES - 1))
      fin_v, fin_i = inf_v, zero_i
      for i in range(16):
        pidx = pbase + i * LANES
        xg = plsc.load_gather(x_v, [pidx])
        yg = plsc.load_gather(y_v, [pidx])
        zg = plsc.load_gather(z_v, [pidx])
        dx = xg - cx
        dy = yg - cy
        dz = zg - cz
        d2 = dx * dx + dy * dy + dz * dz
        fin_v, fin_i = _merge_sorted(fin_v, fin_i, d2, pidx)

      plsc.store_scatter(
          out_v, [lane_iota, jnp.full((LANES,), cl, jnp.int32)], fin_i)

    pltpu.sync_copy(
        out_v, out_h.at[pl.ds(b * KNN, KNN), pl.ds(c0, cols_per_worker)])

  return knn


def kernel(xyz, centers):
  b_sz, n_pts, _ = xyz.shape
  k_cen = centers.shape[1]
  knn = _make_knn(b_sz, n_pts, k_cen)
  pts = jnp.transpose(xyz, (2, 0, 1)).reshape(3, b_sz * n_pts)
  cen = jnp.transpose(centers, (2, 0, 1)).reshape(3, b_sz * k_cen)
  out2d = knn(pts[0], pts[1], pts[2], cen[0], cen[1], cen[2])
  return out2d.reshape(b_sz, KNN, k_cen)
